# baseline (speedup 1.0000x reference)
import jax
import jax.numpy as jnp
from jax import lax
from jax.experimental import pallas as pl
from jax.experimental.pallas import tpu as pltpu
from jax.experimental.pallas import tpu_sc as plsc

VOCAB = 100000
EMB_DIM = 128
HIDDEN = 256
NUM_CLASSES = 100
B = 4096
L = 200

NUM_CORES = 2
NUM_SUBCORES = 16
NW = NUM_CORES * NUM_SUBCORES
G1 = 104
G2 = L - G1
LANES = 16
NV = EMB_DIM // LANES

UNROLL = 8


def _make_pool(nb):
    bpw = nb // NW

    def body(ids_hbm, emb_hbm, out_hbm, idx_v, rows_v, out_v,
             sem0, sem1, sem2, sem3, sem4, sem5):
        c = lax.axis_index("c")
        s = lax.axis_index("s")
        wid = s * NUM_CORES + c
        base = pl.multiple_of(wid * bpw, 8)
        pltpu.sync_copy(
            ids_hbm.at[pl.ds(pl.multiple_of(wid * (bpw * L), 8), bpw * L)],
            idx_v)

        sems = (sem0, sem1, sem2, sem3, sem4, sem5)
        sizes = (G1, G2)

        def issue(b, half, buf):
            off = pl.multiple_of(b * L + half * G1, 8)
            pltpu.async_copy(
                emb_hbm.at[idx_v.at[pl.ds(off, sizes[half])]],
                rows_v.at[buf, pl.ds(0, sizes[half])], sems[buf])

        def drain(half, buf):
            pltpu.make_async_copy(
                emb_hbm.at[pl.ds(0, sizes[half])],
                rows_v.at[buf, pl.ds(0, sizes[half])], sems[buf]).wait()

        def accum(half, buf, acc):
            def acc_body(jj, a):
                for r in range(UNROLL):
                    j = jj * UNROLL + r
                    a = tuple(a[k] + rows_v[buf, j, pl.ds(k * LANES, LANES)]
                              for k in range(NV))
                return a

            return lax.fori_loop(0, sizes[half] // UNROLL, acc_body, acc)

        for r in range(3):
            issue(r, 0, 2 * r)
            issue(r, 1, 2 * r + 1)

        def consume(b, buf0, buf1):
            zeros = tuple(jnp.zeros((LANES,), jnp.float32) for _ in range(NV))
            drain(0, buf0)
            acc = accum(0, buf0, zeros)

            @pl.when(b + 3 < bpw)
            def _issue_a():
                issue(b + 3, 0, buf0)

            drain(1, buf1)
            acc = accum(1, buf1, acc)

            @pl.when(b + 3 < bpw)
            def _issue_b():
                issue(b + 3, 1, buf1)

            for k in range(NV):
                out_v[b, pl.ds(k * LANES, LANES)] = acc[k]

        def trio_body(tt, _):
            for pr in range(3):
                consume(tt * 3 + pr, 2 * pr, 2 * pr + 1)
            return _

        ntrios = bpw // 3
        lax.fori_loop(0, ntrios, trio_body, 0)
        for tail in range(bpw - 3 * ntrios):
            consume(3 * ntrios + tail, 2 * tail, 2 * tail + 1)
        pltpu.sync_copy(out_v, out_hbm.at[pl.ds(base, bpw)])

    mesh = plsc.VectorSubcoreMesh(core_axis_name="c", subcore_axis_name="s")
    return pl.kernel(
        body,
        out_type=jax.ShapeDtypeStruct((nb, EMB_DIM), jnp.float32),
        mesh=mesh,
        scratch_types=[
            pltpu.VMEM((bpw * L,), jnp.int32),
            pltpu.VMEM((6, G1, EMB_DIM), jnp.float32),
            pltpu.VMEM((bpw, EMB_DIM), jnp.float32),
            pltpu.SemaphoreType.DMA,
            pltpu.SemaphoreType.DMA,
            pltpu.SemaphoreType.DMA,
            pltpu.SemaphoreType.DMA,
            pltpu.SemaphoreType.DMA,
            pltpu.SemaphoreType.DMA,
        ],
    )


_HALF = B // 2
_POOL_HALF = _make_pool(_HALF)


def _mlp_body(x_ref, w1_ref, b1_ref, w2_ref, b2_ref, o_ref):
    x = x_ref[...] * (1.0 / L)
    h = jnp.dot(x, w1_ref[...], preferred_element_type=jnp.float32)
    h = jnp.maximum(h + b1_ref[...], 0.0)
    o = jnp.dot(h, w2_ref[...], preferred_element_type=jnp.float32)
    o_ref[...] = o + b2_ref[...]


def _mlp(x, w1, b1, w2, b2):
    nb = x.shape[0]
    bt = 1024
    return pl.pallas_call(
        _mlp_body,
        grid=(nb // bt,),
        in_specs=[
            pl.BlockSpec((bt, EMB_DIM), lambda i: (i, 0)),
            pl.BlockSpec((EMB_DIM, HIDDEN), lambda i: (0, 0)),
            pl.BlockSpec((1, HIDDEN), lambda i: (0, 0)),
            pl.BlockSpec((HIDDEN, NUM_CLASSES), lambda i: (0, 0)),
            pl.BlockSpec((1, NUM_CLASSES), lambda i: (0, 0)),
        ],
        out_specs=pl.BlockSpec((bt, NUM_CLASSES), lambda i: (i, 0)),
        out_shape=jax.ShapeDtypeStruct((nb, NUM_CLASSES), jnp.float32),
    )(x, w1, b1, w2, b2)


def kernel(input_ids, emb, W1, b1, W2, b2):
    ids = input_ids.astype(jnp.int32).reshape(2, _HALF * L)
    b1r = b1.reshape(1, HIDDEN)
    b2r = b2.reshape(1, NUM_CLASSES)
    pooled0 = _POOL_HALF(ids[0], emb)
    pooled1 = _POOL_HALF(ids[1], emb)
    out0 = _mlp(pooled0, W1, b1r, W2, b2r)
    out1 = _mlp(pooled1, W1, b1r, W2, b2r)
    return jnp.concatenate([out0, out1], axis=0)


# speedup vs baseline: 1.0977x; 1.0977x over previous
import jax
import jax.numpy as jnp
from jax import lax
from jax.experimental import pallas as pl
from jax.experimental.pallas import tpu as pltpu
from jax.experimental.pallas import tpu_sc as plsc

VOCAB = 100000
EMB_DIM = 128
HIDDEN = 256
NUM_CLASSES = 100
B = 4096
L = 200

NUM_CORES = 2
NUM_SUBCORES = 16
NW = NUM_CORES * NUM_SUBCORES
G1 = 104
G2 = L - G1
LANES = 16
NV = EMB_DIM // LANES

UNROLL = 8


def _make_pool(nb):
    bpw = nb // NW

    def body(ids_hbm, emb_hbm, out_hbm, idx_v, rows_v, out_v,
             sem0, sem1, sem2, sem3, sem4, sem5):
        c = lax.axis_index("c")
        s = lax.axis_index("s")
        wid = s * NUM_CORES + c
        base = pl.multiple_of(wid * bpw, 8)
        pltpu.sync_copy(
            ids_hbm.at[pl.ds(pl.multiple_of(wid * (bpw * L), 8), bpw * L)],
            idx_v)

        sems = (sem0, sem1, sem2, sem3, sem4, sem5)
        sizes = (G1, G2)

        def issue(b, half, buf):
            off = pl.multiple_of(b * L + half * G1, 8)
            pltpu.async_copy(
                emb_hbm.at[idx_v.at[pl.ds(off, sizes[half])]],
                rows_v.at[buf, pl.ds(0, sizes[half])], sems[buf])

        def drain(half, buf):
            pltpu.make_async_copy(
                emb_hbm.at[pl.ds(0, sizes[half])],
                rows_v.at[buf, pl.ds(0, sizes[half])], sems[buf]).wait()

        def accum(half, buf, acc):
            def acc_body(jj, a):
                for r in range(UNROLL):
                    j = jj * UNROLL + r
                    a = tuple(a[k] + rows_v[buf, j, pl.ds(k * LANES, LANES)]
                              for k in range(NV))
                return a

            return lax.fori_loop(0, sizes[half] // UNROLL, acc_body, acc)

        for r in range(3):
            issue(r, 0, 2 * r)
            issue(r, 1, 2 * r + 1)

        def consume(b, buf0, buf1):
            zeros = tuple(jnp.zeros((LANES,), jnp.float32) for _ in range(NV))
            drain(0, buf0)
            acc = accum(0, buf0, zeros)

            @pl.when(b + 3 < bpw)
            def _issue_a():
                issue(b + 3, 0, buf0)

            drain(1, buf1)
            acc = accum(1, buf1, acc)

            @pl.when(b + 3 < bpw)
            def _issue_b():
                issue(b + 3, 1, buf1)

            for k in range(NV):
                out_v[b, pl.ds(k * LANES, LANES)] = acc[k]

        def trio_body(tt, _):
            for pr in range(3):
                consume(tt * 3 + pr, 2 * pr, 2 * pr + 1)
            return _

        ntrios = bpw // 3
        lax.fori_loop(0, ntrios, trio_body, 0)
        for tail in range(bpw - 3 * ntrios):
            consume(3 * ntrios + tail, 2 * tail, 2 * tail + 1)
        pltpu.sync_copy(out_v, out_hbm.at[pl.ds(base, bpw)])

    mesh = plsc.VectorSubcoreMesh(core_axis_name="c", subcore_axis_name="s")
    return pl.kernel(
        body,
        out_type=jax.ShapeDtypeStruct((nb, EMB_DIM), jnp.float32),
        mesh=mesh,
        scratch_types=[
            pltpu.VMEM((bpw * L,), jnp.int32),
            pltpu.VMEM((6, G1, EMB_DIM), jnp.float32),
            pltpu.VMEM((bpw, EMB_DIM), jnp.float32),
            pltpu.SemaphoreType.DMA,
            pltpu.SemaphoreType.DMA,
            pltpu.SemaphoreType.DMA,
            pltpu.SemaphoreType.DMA,
            pltpu.SemaphoreType.DMA,
            pltpu.SemaphoreType.DMA,
        ],
    )


_POOL = _make_pool(B)


def _mlp_body(x_ref, w1_ref, b1_ref, w2_ref, b2_ref, o_ref):
    x = x_ref[...] * (1.0 / L)
    h = jnp.dot(x, w1_ref[...], preferred_element_type=jnp.float32)
    h = jnp.maximum(h + b1_ref[...], 0.0)
    o = jnp.dot(h, w2_ref[...], preferred_element_type=jnp.float32)
    o_ref[...] = o + b2_ref[...]


def _mlp(x, w1, b1, w2, b2):
    nb = x.shape[0]
    bt = 4096
    return pl.pallas_call(
        _mlp_body,
        grid=(nb // bt,),
        in_specs=[
            pl.BlockSpec((bt, EMB_DIM), lambda i: (i, 0)),
            pl.BlockSpec((EMB_DIM, HIDDEN), lambda i: (0, 0)),
            pl.BlockSpec((1, HIDDEN), lambda i: (0, 0)),
            pl.BlockSpec((HIDDEN, NUM_CLASSES), lambda i: (0, 0)),
            pl.BlockSpec((1, NUM_CLASSES), lambda i: (0, 0)),
        ],
        out_specs=pl.BlockSpec((bt, NUM_CLASSES), lambda i: (i, 0)),
        out_shape=jax.ShapeDtypeStruct((nb, NUM_CLASSES), jnp.float32),
    )(x, w1, b1, w2, b2)


def kernel(input_ids, emb, W1, b1, W2, b2):
    ids = input_ids.astype(jnp.int32).reshape(B * L)
    pooled = _POOL(ids, emb)
    return _mlp(pooled, W1, b1.reshape(1, HIDDEN), W2, b2.reshape(1, NUM_CLASSES))
